# Initial kernel scaffold; baseline (speedup 1.0000x reference)
import jax
import jax.numpy as jnp
from jax.experimental import pallas as pl


def kernel(cat_bboxes, cat_labels):
    raise NotImplementedError("write your pallas kernel here")


# speedup vs baseline: 51.7950x; 51.7950x over previous
import jax
import jax.numpy as jnp
from jax import lax
from jax.experimental import pallas as pl
from jax.experimental.pallas import tpu as pltpu

_NUM_CLASSES = 80
_SCORE_THR = 0.05
_IOU_THR = 0.5
_MAX_PER_IMG = 100
_PRE_NMS = 1000
_CLASS_OFFSET = 4096.0
_N = 20000
_ROWS = 160
_LANES = 128
_NPAD = _ROWS * _LANES


def _nms_body(x1_ref, y1_ref, x2_ref, y2_ref, s_ref, lbl_ref, out_ref,
              sc_ref, x1o_ref, y1o_ref, x2o_ref, y2o_ref, area_ref):
    f32 = jnp.float32
    neg = jnp.array(-jnp.inf, f32)
    s = s_ref[:, :]
    key = jnp.where(s > _SCORE_THR, lax.bitcast_convert_type(s, jnp.int32),
                    jnp.int32(0))
    idx2 = (lax.broadcasted_iota(jnp.int32, (_ROWS, _LANES), 0) * _LANES
            + lax.broadcasted_iota(jnp.int32, (_ROWS, _LANES), 1))

    def bs_body(_, carry):
        lo, hi = carry
        mid = lo + (hi - lo) // 2
        cnt = jnp.sum((key >= mid).astype(jnp.int32))
        ge = cnt >= _PRE_NMS
        return (jnp.where(ge, mid, lo), jnp.where(ge, hi, mid))

    T, _ = lax.fori_loop(0, 31, bs_body,
                         (jnp.int32(0), jnp.int32(0x7F800000)))
    k = jnp.sum((key > T).astype(jnp.int32))
    m = jnp.where(T > 0, _PRE_NMS - k, 0)

    tie = key == T

    def bs2_body(_, carry):
        lo2, hi2 = carry
        mid = lo2 + (hi2 - lo2) // 2
        c = jnp.sum((tie & (idx2 < mid)).astype(jnp.int32))
        ge = c >= m
        return (jnp.where(ge, lo2, mid), jnp.where(ge, mid, hi2))

    _, J = lax.fori_loop(0, 16, bs2_body, (jnp.int32(0), jnp.int32(_NPAD)))
    J = jnp.where(m > 0, J, 0)

    selected = (key > T) | (tie & (T > 0) & (idx2 < J))
    sc_ref[:, :] = jnp.where(selected, s, neg)

    offs = lbl_ref[:, :].astype(f32) * _CLASS_OFFSET
    x1o = x1_ref[:, :] + offs
    y1o = y1_ref[:, :] + offs
    x2o = x2_ref[:, :] + offs
    y2o = y2_ref[:, :] + offs
    x1o_ref[:, :] = x1o
    y1o_ref[:, :] = y1o
    x2o_ref[:, :] = x2o
    y2o_ref[:, :] = y2o
    area_ref[:, :] = (x2o - x1o) * (y2o - y1o)

    row8 = lax.broadcasted_iota(jnp.int32, (8, _LANES), 0)
    col8 = lax.broadcasted_iota(jnp.int32, (8, _LANES), 1)

    def step(t, out):
        scur = sc_ref[:, :]
        mval = jnp.max(scur)
        bidx = jnp.min(jnp.where(scur == mval, idx2, jnp.int32(_NPAD)))
        onehot = idx2 == bidx
        bx1 = jnp.sum(jnp.where(onehot, x1_ref[:, :], 0.0))
        by1 = jnp.sum(jnp.where(onehot, y1_ref[:, :], 0.0))
        bx2 = jnp.sum(jnp.where(onehot, x2_ref[:, :], 0.0))
        by2 = jnp.sum(jnp.where(onehot, y2_ref[:, :], 0.0))
        bl = jnp.sum(jnp.where(onehot, lbl_ref[:, :], jnp.int32(0)))
        blf = bl.astype(f32)
        ox1 = bx1 + blf * _CLASS_OFFSET
        oy1 = by1 + blf * _CLASS_OFFSET
        ox2 = bx2 + blf * _CLASS_OFFSET
        oy2 = by2 + blf * _CLASS_OFFSET
        a1 = (ox2 - ox1) * (oy2 - oy1)
        ix1 = jnp.maximum(ox1, x1o_ref[:, :])
        iy1 = jnp.maximum(oy1, y1o_ref[:, :])
        ix2 = jnp.minimum(ox2, x2o_ref[:, :])
        iy2 = jnp.minimum(oy2, y2o_ref[:, :])
        inter = jnp.maximum(ix2 - ix1, 0.0) * jnp.maximum(iy2 - iy1, 0.0)
        iou = inter / (a1 + area_ref[:, :] - inter + 1e-6)
        ns = jnp.where(iou >= _IOU_THR, neg, scur)
        ns = jnp.where(onehot, neg, ns)
        sc_ref[:, :] = ns
        valid = mval > neg
        vx1 = jnp.where(valid, bx1, 0.0)
        vy1 = jnp.where(valid, by1, 0.0)
        vx2 = jnp.where(valid, bx2, 0.0)
        vy2 = jnp.where(valid, by2, 0.0)
        vsc = jnp.where(valid, mval, 0.0)
        vlb = jnp.where(valid, blf, -1.0)
        newcol = jnp.where(row8 == 0, vx1,
                 jnp.where(row8 == 1, vy1,
                 jnp.where(row8 == 2, vx2,
                 jnp.where(row8 == 3, vy2,
                 jnp.where(row8 == 4, vsc, vlb)))))
        return jnp.where(col8 == t, newcol, out)

    out = lax.fori_loop(0, _MAX_PER_IMG, step, jnp.zeros((8, _LANES), f32))
    out_ref[:, :] = out


def kernel(cat_bboxes, cat_labels):
    pad = _NPAD - _N
    cb = jnp.pad(cat_bboxes, ((0, pad), (0, 0)))
    x1 = cb[:, 0].reshape(_ROWS, _LANES)
    y1 = cb[:, 1].reshape(_ROWS, _LANES)
    x2 = cb[:, 2].reshape(_ROWS, _LANES)
    y2 = cb[:, 3].reshape(_ROWS, _LANES)
    s = cb[:, 4].reshape(_ROWS, _LANES)
    lbl = jnp.pad(cat_labels, (0, pad)).reshape(_ROWS, _LANES)
    out = pl.pallas_call(
        _nms_body,
        out_shape=jax.ShapeDtypeStruct((8, _LANES), jnp.float32),
        scratch_shapes=[pltpu.VMEM((_ROWS, _LANES), jnp.float32)
                        for _ in range(6)],
    )(x1, y1, x2, y2, s, lbl)
    det_bboxes = out[0:5, :_MAX_PER_IMG].T
    det_labels = out[5, :_MAX_PER_IMG].astype(jnp.int32)
    return det_bboxes, det_labels


# speedup vs baseline: 53.5376x; 1.0336x over previous
import jax
import jax.numpy as jnp
from jax import lax
from jax.experimental import pallas as pl
from jax.experimental.pallas import tpu as pltpu

_NUM_CLASSES = 80
_SCORE_THR = 0.05
_IOU_THR = 0.5
_MAX_PER_IMG = 100
_PRE_NMS = 1000
_CLASS_OFFSET = 4096.0
_N = 20000
_ROWS = 160
_LANES = 128
_NPAD = _ROWS * _LANES


def _nms_body(x1_ref, y1_ref, x2_ref, y2_ref, s_ref, lbl_ref, out_ref,
              sc_ref, x1o_ref, y1o_ref, x2o_ref, y2o_ref, area_ref):
    f32 = jnp.float32
    neg = jnp.array(-jnp.inf, f32)
    s = s_ref[:, :]
    key = jnp.where(s > _SCORE_THR, lax.bitcast_convert_type(s, jnp.int32),
                    jnp.int32(0))
    idx2 = (lax.broadcasted_iota(jnp.int32, (_ROWS, _LANES), 0) * _LANES
            + lax.broadcasted_iota(jnp.int32, (_ROWS, _LANES), 1))

    def bs_body(_, carry):
        lo, hi = carry
        mid = lo + (hi - lo) // 2
        cnt = jnp.sum((key >= mid).astype(jnp.int32))
        ge = cnt >= _PRE_NMS
        return (jnp.where(ge, mid, lo), jnp.where(ge, hi, mid))

    T, _ = lax.fori_loop(0, 31, bs_body,
                         (jnp.int32(0), jnp.int32(0x7F800000)))
    k = jnp.sum((key > T).astype(jnp.int32))
    m = jnp.where(T > 0, _PRE_NMS - k, 0)

    tie = key == T

    def bs2_body(_, carry):
        lo2, hi2 = carry
        mid = lo2 + (hi2 - lo2) // 2
        c = jnp.sum((tie & (idx2 < mid)).astype(jnp.int32))
        ge = c >= m
        return (jnp.where(ge, lo2, mid), jnp.where(ge, mid, hi2))

    _, J = lax.fori_loop(0, 16, bs2_body, (jnp.int32(0), jnp.int32(_NPAD)))
    J = jnp.where(m > 0, J, 0)

    selected = (key > T) | (tie & (T > 0) & (idx2 < J))
    sc_ref[:, :] = jnp.where(selected, s, neg)

    offs = lbl_ref[:, :].astype(f32) * _CLASS_OFFSET
    x1o = x1_ref[:, :] + offs
    y1o = y1_ref[:, :] + offs
    x2o = x2_ref[:, :] + offs
    y2o = y2_ref[:, :] + offs
    x1o_ref[:, :] = x1o
    y1o_ref[:, :] = y1o
    x2o_ref[:, :] = x2o
    y2o_ref[:, :] = y2o
    area_ref[:, :] = (x2o - x1o) * (y2o - y1o)

    row8 = lax.broadcasted_iota(jnp.int32, (8, _LANES), 0)
    col8 = lax.broadcasted_iota(jnp.int32, (8, _LANES), 1)

    def step(t, out):
        scur = sc_ref[:, :]
        mval = jnp.max(scur)
        bidx = jnp.min(jnp.where(scur == mval, idx2, jnp.int32(_NPAD)))
        onehot = idx2 == bidx
        br = bidx // _LANES
        bc = bidx % _LANES
        lane1 = lax.broadcasted_iota(jnp.int32, (1, _LANES), 1)
        lhot = lane1 == bc

        def ext_f(ref):
            return jnp.sum(jnp.where(lhot, ref[pl.ds(br, 1), :], 0.0))

        bx1 = ext_f(x1_ref)
        by1 = ext_f(y1_ref)
        bx2 = ext_f(x2_ref)
        by2 = ext_f(y2_ref)
        bl = jnp.sum(jnp.where(lhot, lbl_ref[pl.ds(br, 1), :], jnp.int32(0)))
        blf = bl.astype(f32)
        ox1 = bx1 + blf * _CLASS_OFFSET
        oy1 = by1 + blf * _CLASS_OFFSET
        ox2 = bx2 + blf * _CLASS_OFFSET
        oy2 = by2 + blf * _CLASS_OFFSET
        a1 = (ox2 - ox1) * (oy2 - oy1)
        ix1 = jnp.maximum(ox1, x1o_ref[:, :])
        iy1 = jnp.maximum(oy1, y1o_ref[:, :])
        ix2 = jnp.minimum(ox2, x2o_ref[:, :])
        iy2 = jnp.minimum(oy2, y2o_ref[:, :])
        inter = jnp.maximum(ix2 - ix1, 0.0) * jnp.maximum(iy2 - iy1, 0.0)
        iou = inter / (a1 + area_ref[:, :] - inter + 1e-6)
        ns = jnp.where(iou >= _IOU_THR, neg, scur)
        ns = jnp.where(onehot, neg, ns)
        sc_ref[:, :] = ns
        valid = mval > neg
        vx1 = jnp.where(valid, bx1, 0.0)
        vy1 = jnp.where(valid, by1, 0.0)
        vx2 = jnp.where(valid, bx2, 0.0)
        vy2 = jnp.where(valid, by2, 0.0)
        vsc = jnp.where(valid, mval, 0.0)
        vlb = jnp.where(valid, blf, -1.0)
        newcol = jnp.where(row8 == 0, vx1,
                 jnp.where(row8 == 1, vy1,
                 jnp.where(row8 == 2, vx2,
                 jnp.where(row8 == 3, vy2,
                 jnp.where(row8 == 4, vsc, vlb)))))
        return jnp.where(col8 == t, newcol, out)

    out = lax.fori_loop(0, _MAX_PER_IMG, step, jnp.zeros((8, _LANES), f32))
    out_ref[:, :] = out


def kernel(cat_bboxes, cat_labels):
    pad = _NPAD - _N
    cb = jnp.pad(cat_bboxes, ((0, pad), (0, 0)))
    x1 = cb[:, 0].reshape(_ROWS, _LANES)
    y1 = cb[:, 1].reshape(_ROWS, _LANES)
    x2 = cb[:, 2].reshape(_ROWS, _LANES)
    y2 = cb[:, 3].reshape(_ROWS, _LANES)
    s = cb[:, 4].reshape(_ROWS, _LANES)
    lbl = jnp.pad(cat_labels, (0, pad)).reshape(_ROWS, _LANES)
    out = pl.pallas_call(
        _nms_body,
        out_shape=jax.ShapeDtypeStruct((8, _LANES), jnp.float32),
        scratch_shapes=[pltpu.VMEM((_ROWS, _LANES), jnp.float32)
                        for _ in range(6)],
    )(x1, y1, x2, y2, s, lbl)
    det_bboxes = out[0:5, :_MAX_PER_IMG].T
    det_labels = out[5, :_MAX_PER_IMG].astype(jnp.int32)
    return det_bboxes, det_labels
